# baseline (speedup 1.0000x reference)
import functools

import jax
import jax.numpy as jnp
from jax import lax
from jax.experimental import pallas as pl
from jax.experimental.pallas import tpu as pltpu
from jax.experimental.pallas import tpu_sc as plsc

N_FACTORS = 64
BATCH = 16384
NU = BATCH
NT = 2 * BATCH
NC = 2
NS = 16
LANES = 16
NW = NC * NS
CHUNKS = N_FACTORS // LANES
NBLK = 7813
SBW = 4
NSB = (NBLK + SBW - 1) // SBW
SB_PER_W = (NSB + NW - 1) // NW
LAST_SB = NSB - 1
PCAP = 4096
MCAP = 4096


def _sc(ref, pos):
    return ref[pl.ds(pos, LANES)][0]


def _stage1_body(user2, itemi2, itemj2, ut, itt, gu3, gt3,
                 idx_v, rbuf, pbuf, mbuf, mpbuf, blk0, blk1, blk2, rowst,
                 sb0, sb1, sb2, so):
    wid = lax.axis_index("s") * NC + lax.axis_index("c")
    sb_lo = wid * SB_PER_W
    sb_hi = jnp.minimum(sb_lo + SB_PER_W, NSB)
    nq = sb_hi - sb_lo
    iota = lax.iota(jnp.int32, LANES)

    def scan(src2, pos_base, cnt0):
        cnt = cnt0
        for hh in range(2):
            pltpu.sync_copy(src2.at[pl.ds(hh * 64, 64)], idx_v)

            def chunk(k, cnt, _hh=hh):
                rv = idx_v[k >> 3, pl.ds((k & 7) * LANES, LANES)]
                sv = rv >> 9
                m = (sv >= sb_lo) & (sv < sb_hi)
                plsc.store_compressed(rbuf.at[pl.ds(cnt, LANES)], rv, mask=m)
                pv = iota + (k * LANES + (pos_base + _hh * 8192))
                plsc.store_compressed(pbuf.at[pl.ds(cnt, LANES)], pv, mask=m)
                pc = plsc.all_reduce_population_count(m)[0]
                return jnp.minimum(cnt + pc, PCAP - LANES)

            cnt = lax.fori_loop(0, BATCH // LANES // 2, chunk, cnt)
        return cnt

    def fetch(tbl, sb, blk, sem):
        c0 = sb * 128 * SBW

        @pl.when(sb != LAST_SB)
        def _():
            for a in range(8):
                pltpu.async_copy(
                    tbl.at[pl.ds(a * 8, 8),
                           pl.ds(pl.multiple_of(c0, 128), 128 * SBW)],
                    blk.at[pl.ds(a * 8, 8)], sem)

        @pl.when(sb == LAST_SB)
        def _():
            for a in range(8):
                pltpu.async_copy(
                    tbl.at[pl.ds(a * 8, 8),
                           pl.ds(pl.multiple_of(c0, 128), 128)],
                    blk.at[pl.ds(a * 8, 8), pl.ds(0, 128)], sem)

    def wait_fetch(tbl, sb, blk, sem):
        @pl.when(sb != LAST_SB)
        def _():
            for a in range(8):
                pltpu.make_async_copy(
                    tbl.at[pl.ds(0, 8), pl.ds(0, 128 * SBW)],
                    blk.at[pl.ds(a * 8, 8)], sem).wait()

        @pl.when(sb == LAST_SB)
        def _():
            for a in range(8):
                pltpu.make_async_copy(
                    tbl.at[pl.ds(0, 8), pl.ds(0, 128)],
                    blk.at[pl.ds(a * 8, 8), pl.ds(0, 128)], sem).wait()

    def do_sb(sb, blk, npairs, gcnt, gout):
        def mchunk(k, mcnt):
            base = k * LANES
            rv = rbuf[pl.ds(base, LANES)]
            m = (rv >> 9) == sb
            plsc.store_compressed(mbuf.at[pl.ds(mcnt, LANES)], rv, mask=m)
            pv = pbuf[pl.ds(base, LANES)]
            plsc.store_compressed(mpbuf.at[pl.ds(mcnt, LANES)], pv, mask=m)
            pc = plsc.all_reduce_population_count(m)[0]
            return jnp.minimum(mcnt + pc, MCAP - LANES)

        nchunks = (npairs + LANES - 1) // LANES
        mcnt = lax.fori_loop(0, nchunks, mchunk, 0)

        def ext(t, g):
            r = _sc(mbuf, t)
            pos = _sc(mpbuf, t)
            l = (r & 127) + 128 * ((r >> 7) & (SBW - 1))
            slot = g & 7

            @pl.when(g >= 8)
            def _():
                pltpu.make_async_copy(
                    rowst.at[0], gout.at[0], so).wait()
            for kk in range(CHUNKS):
                gth = plsc.load_gather(
                    blk, [iota + kk * LANES, jnp.full((LANES,), l, jnp.int32)])
                rowst[slot, 0, pl.ds(kk * LANES, LANES)] = gth
            pltpu.async_copy(rowst.at[slot], gout.at[pos], so)
            return g + 1

        return lax.fori_loop(0, mcnt, ext, gcnt)

    def run_pass(tbl, npairs, gout):
        plsc.store_compressed(rbuf.at[pl.ds(npairs, LANES)],
                              jnp.full((LANES,), -1, jnp.int32),
                              mask=jnp.full((LANES,), True))

        bufs = (blk0, blk1, blk2)
        sems = (sb0, sb1, sb2)
        fetch(tbl, sb_lo, blk0, sb0)

        @pl.when(1 < nq)
        def _():
            fetch(tbl, sb_lo + 1, blk1, sb1)

        def qbody(q3, gcnt):
            q = 3 * q3
            for i in range(3):
                def step(gc, _i=i):
                    sb = sb_lo + q + _i
                    wait_fetch(tbl, sb, bufs[_i], sems[_i])

                    @pl.when(q + _i + 2 < nq)
                    def _():
                        fetch(tbl, sb + 2, bufs[(_i + 2) % 3],
                              sems[(_i + 2) % 3])
                    return do_sb(sb, bufs[_i], npairs, gc, gout)

                gcnt = lax.cond(q + i < nq, step, lambda gc: gc, gcnt)
            return gcnt

        gcnt = lax.fori_loop(0, (nq + 2) // 3, qbody, 0)

        def drain(i, c):
            pltpu.make_async_copy(rowst.at[0], gout.at[0], so).wait()
            return c

        lax.fori_loop(0, jnp.minimum(gcnt, 8), drain, 0)

    nu_pairs = scan(user2, 0, 0)
    run_pass(ut, nu_pairs, gu3)
    nt_pairs = scan(itemi2, 0, 0)
    nt_pairs = scan(itemj2, BATCH, nt_pairs)
    run_pass(itt, nt_pairs, gt3)


@functools.partial(
    pl.kernel,
    mesh=plsc.VectorSubcoreMesh(core_axis_name="c", subcore_axis_name="s"),
    out_type=(jax.ShapeDtypeStruct((NU, 1, 128), jnp.float32),
              jax.ShapeDtypeStruct((NT, 1, 128), jnp.float32)),
    compiler_params=pltpu.CompilerParams(needs_layout_passes=False),
    scratch_types=[
        pltpu.VMEM((64, 128), jnp.int32),
        pltpu.VMEM((PCAP,), jnp.int32),
        pltpu.VMEM((PCAP,), jnp.int32),
        pltpu.VMEM((MCAP,), jnp.int32),
        pltpu.VMEM((MCAP,), jnp.int32),
        pltpu.VMEM((64, 128 * SBW), jnp.float32),
        pltpu.VMEM((64, 128 * SBW), jnp.float32),
        pltpu.VMEM((64, 128 * SBW), jnp.float32),
        pltpu.VMEM((8, 1, 128), jnp.float32),
        pltpu.SemaphoreType.DMA,
        pltpu.SemaphoreType.DMA,
        pltpu.SemaphoreType.DMA,
        pltpu.SemaphoreType.DMA,
    ],
)
def _stage1(*args):
    _stage1_body(*args)


def _stage2_body(gu3, gt3, out, u3, vi3, vj3, out_v, sem):
    wid = lax.axis_index("s") * NC + lax.axis_index("c")
    lane = lax.iota(jnp.int32, LANES)
    for h in range(2):
        base = wid * 512 + h * 256
        cps = [pltpu.async_copy(gu3.at[pl.ds(base, 256)], u3, sem),
               pltpu.async_copy(gt3.at[pl.ds(base, 256)], vi3, sem),
               pltpu.async_copy(gt3.at[pl.ds(BATCH + base, 256)], vj3, sem)]
        for cp in cps:
            cp.wait()

        def group_body(g, carry):
            gb = g * LANES
            res = jnp.zeros((LANES,), jnp.float32)
            for i in range(LANES):
                r = gb + i
                acc = jnp.zeros((LANES,), jnp.float32)
                for c in range(CHUNKS):
                    sl = pl.ds(c * LANES, LANES)
                    acc = acc + u3[r, 0, sl] * (vi3[r, 0, sl] - vj3[r, 0, sl])
                res = jnp.where(lane == i, jnp.sum(acc), res)
            out_v[pl.ds(gb, LANES)] = res
            return carry

        lax.fori_loop(0, 256 // LANES, group_body, 0)
        pltpu.sync_copy(out_v, out.at[pl.ds(base, 256)])


@functools.partial(
    pl.kernel,
    mesh=plsc.VectorSubcoreMesh(core_axis_name="c", subcore_axis_name="s"),
    out_type=jax.ShapeDtypeStruct((BATCH,), jnp.float32),
    compiler_params=pltpu.CompilerParams(needs_layout_passes=False),
    scratch_types=[
        pltpu.VMEM((256, 1, 128), jnp.float32),
        pltpu.VMEM((256, 1, 128), jnp.float32),
        pltpu.VMEM((256, 1, 128), jnp.float32),
        pltpu.VMEM((256,), jnp.float32),
        pltpu.SemaphoreType.DMA,
    ],
)
def _stage2(*args):
    _stage2_body(*args)


def kernel(user, item_i, item_j, embed_user_w, embed_item_w):
    user2 = user.astype(jnp.int32).reshape(128, 128)
    itemi2 = item_i.astype(jnp.int32).reshape(128, 128)
    itemj2 = item_j.astype(jnp.int32).reshape(128, 128)
    gu3, gt3 = _stage1(user2, itemi2, itemj2,
                       embed_user_w.T, embed_item_w.T)
    return _stage2(gu3, gt3)


# speedup vs baseline: 1.0126x; 1.0126x over previous
import functools

import jax
import jax.numpy as jnp
from jax import lax
from jax.experimental import pallas as pl
from jax.experimental.pallas import tpu as pltpu
from jax.experimental.pallas import tpu_sc as plsc

N_FACTORS = 64
BATCH = 16384
NU = BATCH
NT = 2 * BATCH
NC = 2
NS = 16
LANES = 16
NW = NC * NS
CHUNKS = N_FACTORS // LANES
NBLK = 7813
SBW = 4
NSB = (NBLK + SBW - 1) // SBW
SB_PER_W = (NSB + NW - 1) // NW
LAST_SB = NSB - 1
PCAP = 4096
MCAP = 4096


def _sc(ref, pos):
    return ref[pl.ds(pos, LANES)][0]


def _stage1_body(user2, itemi2, itemj2, ut, itt, gu3, gt3,
                 idx_v, rbuf, pbuf, mbuf, mpbuf, blk0, blk1, blk2, rowst,
                 sb0, sb1, sb2, so):
    wid = lax.axis_index("s") * NC + lax.axis_index("c")
    sb_lo = wid * SB_PER_W
    sb_hi = jnp.minimum(sb_lo + SB_PER_W, NSB)
    nq = sb_hi - sb_lo
    iota = lax.iota(jnp.int32, LANES)

    def scan(src2, pos_base, cnt0):
        cnt = cnt0
        for hh in range(2):
            pltpu.sync_copy(src2.at[pl.ds(hh * 64, 64)], idx_v)

            def chunk(k, cnt, _hh=hh):
                rv = idx_v[k >> 3, pl.ds((k & 7) * LANES, LANES)]
                sv = rv >> 9
                m = (sv >= sb_lo) & (sv < sb_hi)
                plsc.store_compressed(rbuf.at[pl.ds(cnt, LANES)], rv, mask=m)
                pv = iota + (k * LANES + (pos_base + _hh * 8192))
                plsc.store_compressed(pbuf.at[pl.ds(cnt, LANES)], pv, mask=m)
                pc = plsc.all_reduce_population_count(m)[0]
                return jnp.minimum(cnt + pc, PCAP - LANES)

            cnt = lax.fori_loop(0, BATCH // LANES // 2, chunk, cnt,
                                unroll=4)
        return cnt

    def fetch(tbl, sb, blk, sem):
        c0 = sb * 128 * SBW

        @pl.when(sb != LAST_SB)
        def _():
            pltpu.async_copy(
                tbl.at[:, pl.ds(pl.multiple_of(c0, 128), 128 * SBW)],
                blk, sem)

        @pl.when(sb == LAST_SB)
        def _():
            pltpu.async_copy(
                tbl.at[:, pl.ds(pl.multiple_of(c0, 128), 128)],
                blk.at[:, pl.ds(0, 128)], sem)

    def wait_fetch(tbl, sb, blk, sem):
        @pl.when(sb != LAST_SB)
        def _():
            pltpu.make_async_copy(
                tbl.at[:, pl.ds(0, 128 * SBW)], blk, sem).wait()

        @pl.when(sb == LAST_SB)
        def _():
            pltpu.make_async_copy(
                tbl.at[:, pl.ds(0, 128)], blk.at[:, pl.ds(0, 128)],
                sem).wait()

    def do_sb(sb, blk, npairs, gcnt, gout):
        def mchunk(k, mcnt):
            base = k * LANES
            rv = rbuf[pl.ds(base, LANES)]
            m = (rv >> 9) == sb
            plsc.store_compressed(mbuf.at[pl.ds(mcnt, LANES)], rv, mask=m)
            pv = pbuf[pl.ds(base, LANES)]
            plsc.store_compressed(mpbuf.at[pl.ds(mcnt, LANES)], pv, mask=m)
            pc = plsc.all_reduce_population_count(m)[0]
            return jnp.minimum(mcnt + pc, MCAP - LANES)

        nchunks = (npairs + LANES - 1) // LANES
        mcnt = lax.fori_loop(0, nchunks, mchunk, 0)

        def ext(t, g):
            r = _sc(mbuf, t)
            pos = _sc(mpbuf, t)
            l = (r & 127) + 128 * ((r >> 7) & (SBW - 1))
            slot = g & 7

            @pl.when(g >= 8)
            def _():
                pltpu.make_async_copy(
                    rowst.at[0], gout.at[0], so).wait()
            for kk in range(CHUNKS):
                gth = plsc.load_gather(
                    blk, [iota + kk * LANES, jnp.full((LANES,), l, jnp.int32)])
                rowst[slot, 0, pl.ds(kk * LANES, LANES)] = gth
            pltpu.async_copy(rowst.at[slot], gout.at[pos], so)
            return g + 1

        return lax.fori_loop(0, mcnt, ext, gcnt)

    def run_pass(tbl, npairs, gout):
        plsc.store_compressed(rbuf.at[pl.ds(npairs, LANES)],
                              jnp.full((LANES,), -1, jnp.int32),
                              mask=jnp.full((LANES,), True))

        bufs = (blk0, blk1, blk2)
        sems = (sb0, sb1, sb2)
        fetch(tbl, sb_lo, blk0, sb0)

        @pl.when(1 < nq)
        def _():
            fetch(tbl, sb_lo + 1, blk1, sb1)

        def qbody(q3, gcnt):
            q = 3 * q3
            for i in range(3):
                def step(gc, _i=i):
                    sb = sb_lo + q + _i
                    wait_fetch(tbl, sb, bufs[_i], sems[_i])

                    @pl.when(q + _i + 2 < nq)
                    def _():
                        fetch(tbl, sb + 2, bufs[(_i + 2) % 3],
                              sems[(_i + 2) % 3])
                    return do_sb(sb, bufs[_i], npairs, gc, gout)

                gcnt = lax.cond(q + i < nq, step, lambda gc: gc, gcnt)
            return gcnt

        gcnt = lax.fori_loop(0, (nq + 2) // 3, qbody, 0)

        def drain(i, c):
            pltpu.make_async_copy(rowst.at[0], gout.at[0], so).wait()
            return c

        lax.fori_loop(0, jnp.minimum(gcnt, 8), drain, 0)

    nu_pairs = scan(user2, 0, 0)
    run_pass(ut, nu_pairs, gu3)
    nt_pairs = scan(itemi2, 0, 0)
    nt_pairs = scan(itemj2, BATCH, nt_pairs)
    run_pass(itt, nt_pairs, gt3)


@functools.partial(
    pl.kernel,
    mesh=plsc.VectorSubcoreMesh(core_axis_name="c", subcore_axis_name="s"),
    out_type=(jax.ShapeDtypeStruct((NU, 1, 128), jnp.float32),
              jax.ShapeDtypeStruct((NT, 1, 128), jnp.float32)),
    compiler_params=pltpu.CompilerParams(needs_layout_passes=False),
    scratch_types=[
        pltpu.VMEM((64, 128), jnp.int32),
        pltpu.VMEM((PCAP,), jnp.int32),
        pltpu.VMEM((PCAP,), jnp.int32),
        pltpu.VMEM((MCAP,), jnp.int32),
        pltpu.VMEM((MCAP,), jnp.int32),
        pltpu.VMEM((64, 128 * SBW), jnp.float32),
        pltpu.VMEM((64, 128 * SBW), jnp.float32),
        pltpu.VMEM((64, 128 * SBW), jnp.float32),
        pltpu.VMEM((8, 1, 128), jnp.float32),
        pltpu.SemaphoreType.DMA,
        pltpu.SemaphoreType.DMA,
        pltpu.SemaphoreType.DMA,
        pltpu.SemaphoreType.DMA,
    ],
)
def _stage1(*args):
    _stage1_body(*args)


def _stage2_body(gu3, gt3, out, u3, vi3, vj3, out_v, sem):
    wid = lax.axis_index("s") * NC + lax.axis_index("c")
    lane = lax.iota(jnp.int32, LANES)
    for h in range(2):
        base = wid * 512 + h * 256
        cps = [pltpu.async_copy(gu3.at[pl.ds(base, 256)], u3, sem),
               pltpu.async_copy(gt3.at[pl.ds(base, 256)], vi3, sem),
               pltpu.async_copy(gt3.at[pl.ds(BATCH + base, 256)], vj3, sem)]
        for cp in cps:
            cp.wait()

        def group_body(g, carry):
            gb = g * LANES
            res = jnp.zeros((LANES,), jnp.float32)
            for i in range(LANES):
                r = gb + i
                acc = jnp.zeros((LANES,), jnp.float32)
                for c in range(CHUNKS):
                    sl = pl.ds(c * LANES, LANES)
                    acc = acc + u3[r, 0, sl] * (vi3[r, 0, sl] - vj3[r, 0, sl])
                res = jnp.where(lane == i, jnp.sum(acc), res)
            out_v[pl.ds(gb, LANES)] = res
            return carry

        lax.fori_loop(0, 256 // LANES, group_body, 0, unroll=2)
        pltpu.sync_copy(out_v, out.at[pl.ds(base, 256)])


@functools.partial(
    pl.kernel,
    mesh=plsc.VectorSubcoreMesh(core_axis_name="c", subcore_axis_name="s"),
    out_type=jax.ShapeDtypeStruct((BATCH,), jnp.float32),
    compiler_params=pltpu.CompilerParams(needs_layout_passes=False),
    scratch_types=[
        pltpu.VMEM((256, 1, 128), jnp.float32),
        pltpu.VMEM((256, 1, 128), jnp.float32),
        pltpu.VMEM((256, 1, 128), jnp.float32),
        pltpu.VMEM((256,), jnp.float32),
        pltpu.SemaphoreType.DMA,
    ],
)
def _stage2(*args):
    _stage2_body(*args)


def kernel(user, item_i, item_j, embed_user_w, embed_item_w):
    user2 = user.astype(jnp.int32).reshape(128, 128)
    itemi2 = item_i.astype(jnp.int32).reshape(128, 128)
    itemj2 = item_j.astype(jnp.int32).reshape(128, 128)
    gu3, gt3 = _stage1(user2, itemi2, itemj2,
                       embed_user_w.T, embed_item_w.T)
    return _stage2(gu3, gt3)
